# baseline (speedup 1.0000x reference)
import jax
import jax.numpy as jnp
from jax import lax
from jax.experimental import pallas as pl
from jax.experimental.pallas import tpu as pltpu

MEM = 20000
FEAT = 3 * 32 * 32
NCLS = 100
PK = NCLS + 2
BATCH = 4096
NCHUNK = 10
CROWS = MEM // NCHUNK


def _body(idx_ref, xf_hbm, pk_hbm, bx_hbm, pkb_hbm, obx_hbm, opk_hbm,
          winner_ref, sem_bulk, sem_x, sem_pk):
    for c in range(NCHUNK):
        sl = pl.ds(c * CROWS, CROWS)
        pltpu.make_async_copy(bx_hbm.at[sl], obx_hbm.at[sl], sem_bulk).start()
        pltpu.make_async_copy(pkb_hbm.at[sl], opk_hbm.at[sl], sem_bulk).start()

    @pl.loop(0, MEM)
    def _(s):
        winner_ref[s] = -1

    @pl.loop(0, BATCH)
    def _(i):
        winner_ref[idx_ref[i]] = i

    for c in range(NCHUNK):
        sl = pl.ds(c * CROWS, CROWS)
        pltpu.make_async_copy(bx_hbm.at[sl], obx_hbm.at[sl], sem_bulk).wait()
        pltpu.make_async_copy(pkb_hbm.at[sl], opk_hbm.at[sl], sem_bulk).wait()

    def issue(i, cnt):
        s = idx_ref[i]

        @pl.when(winner_ref[s] == i)
        def _():
            pltpu.make_async_copy(
                xf_hbm.at[pl.ds(i, 1)], obx_hbm.at[pl.ds(s, 1)], sem_x).start()
            pltpu.make_async_copy(
                pk_hbm.at[pl.ds(i, 1)], opk_hbm.at[pl.ds(s, 1)], sem_pk).start()

        return cnt + jnp.where(winner_ref[s] == i, 1, 0)

    cnt = lax.fori_loop(0, BATCH, issue, 0)

    def drain(_, carry):
        pltpu.make_async_copy(
            xf_hbm.at[pl.ds(0, 1)], obx_hbm.at[pl.ds(0, 1)], sem_x).wait()
        pltpu.make_async_copy(
            pk_hbm.at[pl.ds(0, 1)], opk_hbm.at[pl.ds(0, 1)], sem_pk).wait()
        return carry

    lax.fori_loop(0, cnt, drain, 0)


def kernel(x, y, logits, t, idx, bx, by, bt, blogits):
    xf = x.reshape(BATCH, FEAT)
    bxf = bx.reshape(MEM, FEAT)

    logits_bits = jax.lax.bitcast_convert_type(logits, jnp.int32)
    t_col = jnp.full((BATCH, 1), t, dtype=jnp.int32)
    pk_in = jnp.concatenate([logits_bits, y[:, None], t_col], axis=1)

    blogits_bits = jax.lax.bitcast_convert_type(blogits, jnp.int32)
    pk_buf = jnp.concatenate([blogits_bits, by[:, None], bt[:, None]], axis=1)

    obx, opk = pl.pallas_call(
        _body,
        grid_spec=pltpu.PrefetchScalarGridSpec(
            num_scalar_prefetch=1,
            grid=(1,),
            in_specs=[
                pl.BlockSpec(memory_space=pl.ANY),
                pl.BlockSpec(memory_space=pl.ANY),
                pl.BlockSpec(memory_space=pl.ANY),
                pl.BlockSpec(memory_space=pl.ANY),
            ],
            out_specs=[
                pl.BlockSpec(memory_space=pl.ANY),
                pl.BlockSpec(memory_space=pl.ANY),
            ],
            scratch_shapes=[
                pltpu.SMEM((MEM,), jnp.int32),
                pltpu.SemaphoreType.DMA,
                pltpu.SemaphoreType.DMA,
                pltpu.SemaphoreType.DMA,
            ],
        ),
        out_shape=[
            jax.ShapeDtypeStruct((MEM, FEAT), jnp.float32),
            jax.ShapeDtypeStruct((MEM, PK), jnp.int32),
        ],
    )(idx, xf, pk_in, bxf, pk_buf)

    bx_new = obx.reshape(MEM, 3, 32, 32)
    blogits_new = jax.lax.bitcast_convert_type(opk[:, :NCLS], jnp.float32)
    by_new = opk[:, NCLS]
    bt_new = opk[:, NCLS + 1]
    return (bx_new, by_new, bt_new, blogits_new)


# speedup vs baseline: 12.5090x; 12.5090x over previous
import jax
import jax.numpy as jnp
from jax import lax
from jax.experimental import pallas as pl
from jax.experimental.pallas import tpu as pltpu
from jax.experimental.pallas import tpu_sc as plsc

MEM = 20000
FEAT = 3 * 32 * 32
NCLS = 100
PK = 128
BATCH = 4096
COPY_ROWS = 512
KCHUNK = 512
NSUB = 32
WROWS = BATCH // NSUB
XSUB = 32

_vector_mesh = plsc.VectorSubcoreMesh(
    core_axis_name="core", subcore_axis_name="subcore")


def _kmap_body(idx_col_ref, idx_row_ref, out_ref):
    own = idx_col_ref[...]
    allv = idx_row_ref[...]
    iota = lax.broadcasted_iota(jnp.int32, (KCHUNK, BATCH), 1)
    sel = jnp.where(own == allv, iota, -1)
    out_ref[...] = jnp.max(sel, axis=1, keepdims=True)


def _copy_body(bx_ref, pk_ref, obx_ref, opk_ref):
    obx_ref[...] = bx_ref[...]
    opk_ref[...] = pk_ref[...]


def _make_sc_scatter():
    def body(idx_hbm, kmap_hbm, x_hbm, pkin_hbm, obx_ref, opk_ref,
             iw_vmem, kw_vmem, xw_vmem, pkw_vmem):
        core = lax.axis_index("core")
        sub = lax.axis_index("subcore")
        off = (core * 16 + sub) * WROWS

        pltpu.sync_copy(idx_hbm.at[0, pl.ds(off, WROWS)], iw_vmem)
        pltpu.sync_copy(kmap_hbm.at[0, pl.ds(off, WROWS)], kw_vmem)

        pltpu.sync_copy(pkin_hbm.at[kw_vmem], pkw_vmem)
        pltpu.sync_copy(pkw_vmem, opk_ref.at[iw_vmem])

        for k in range(WROWS // XSUB):
            sl = pl.ds(k * XSUB, XSUB)
            pltpu.sync_copy(x_hbm.at[kw_vmem.at[sl]], xw_vmem)
            pltpu.sync_copy(xw_vmem, obx_ref.at[iw_vmem.at[sl]])

    return pl.kernel(
        body,
        out_type=(),
        mesh=_vector_mesh,
        scratch_types=[
            pltpu.VMEM((WROWS,), jnp.int32),
            pltpu.VMEM((WROWS,), jnp.int32),
            pltpu.VMEM((XSUB, FEAT), jnp.float32),
            pltpu.VMEM((WROWS, PK), jnp.int32),
        ],
    )


def kernel(x, y, logits, t, idx, bx, by, bt, blogits):
    xf = x.reshape(BATCH, FEAT)
    bxf = bx.reshape(MEM, FEAT)

    logits_bits = jax.lax.bitcast_convert_type(logits, jnp.int32)
    t_col = jnp.full((BATCH, 1), t, dtype=jnp.int32)
    pad_in = jnp.zeros((BATCH, PK - NCLS - 2), jnp.int32)
    pk_in = jnp.concatenate([logits_bits, y[:, None], t_col, pad_in], axis=1)

    blogits_bits = jax.lax.bitcast_convert_type(blogits, jnp.int32)
    pad_buf = jnp.zeros((MEM, PK - NCLS - 2), jnp.int32)
    pk_buf = jnp.concatenate(
        [blogits_bits, by[:, None], bt[:, None], pad_buf], axis=1)

    kmap = pl.pallas_call(
        _kmap_body,
        grid=(BATCH // KCHUNK,),
        in_specs=[
            pl.BlockSpec((KCHUNK, 1), lambda i: (i, 0)),
            pl.BlockSpec((1, BATCH), lambda i: (0, 0)),
        ],
        out_specs=pl.BlockSpec((KCHUNK, 1), lambda i: (i, 0)),
        out_shape=jax.ShapeDtypeStruct((BATCH, 1), jnp.int32),
    )(idx[:, None], idx[None, :])

    cbx, cpk = pl.pallas_call(
        _copy_body,
        grid=(pl.cdiv(MEM, COPY_ROWS),),
        in_specs=[
            pl.BlockSpec((COPY_ROWS, FEAT), lambda i: (i, 0)),
            pl.BlockSpec((COPY_ROWS, PK), lambda i: (i, 0)),
        ],
        out_specs=[
            pl.BlockSpec((COPY_ROWS, FEAT), lambda i: (i, 0)),
            pl.BlockSpec((COPY_ROWS, PK), lambda i: (i, 0)),
        ],
        out_shape=[
            jax.ShapeDtypeStruct((MEM, FEAT), jnp.float32),
            jax.ShapeDtypeStruct((MEM, PK), jnp.int32),
        ],
    )(bxf, pk_buf)

    obx_ref = jax.new_ref(cbx)
    opk_ref = jax.new_ref(cpk)
    _make_sc_scatter()(idx[None, :], kmap.reshape(1, BATCH), xf, pk_in,
                       obx_ref, opk_ref)
    obx = obx_ref[...]
    opk = opk_ref[...]

    bx_new = obx.reshape(MEM, 3, 32, 32)
    blogits_new = jax.lax.bitcast_convert_type(opk[:, :NCLS], jnp.float32)
    by_new = opk[:, NCLS]
    bt_new = opk[:, NCLS + 1]
    return (bx_new, by_new, bt_new, blogits_new)
